# baseline (speedup 1.0000x reference)
import functools

import jax
import jax.numpy as jnp
from jax import lax
from jax.experimental import pallas as pl
from jax.experimental.pallas import tpu as pltpu
from jax.experimental.pallas import tpu_sc as plsc

B, L, D = 16, 2048, 256
LANES = 16
NW = 32
RPW = (B * L) // NW
CHUNK = 128
NCH = RPW // CHUNK
NBUF = 3

_SC_MESH = plsc.VectorSubcoreMesh(core_axis_name="c", subcore_axis_name="s")


@functools.partial(
    pl.kernel,
    out_type=jax.ShapeDtypeStruct((B * L, D), jnp.float32),
    mesh=_SC_MESH,
    compiler_params=pltpu.CompilerParams(needs_layout_passes=False),
    scratch_types=[
        pltpu.VMEM((L,), jnp.int32),
        pltpu.VMEM((L,), jnp.int32),
        pltpu.VMEM((RPW,), jnp.int32),
        pltpu.VMEM((NBUF, CHUNK, D), jnp.float32),
        pltpu.SemaphoreType.DMA,
        pltpu.SemaphoreType.DMA,
        pltpu.SemaphoreType.DMA,
        pltpu.SemaphoreType.DMA,
        pltpu.SemaphoreType.DMA,
        pltpu.SemaphoreType.DMA,
        pltpu.SemaphoreType.DMA,
        pltpu.SemaphoreType.DMA,
    ],
)
def _sc_flatten(mask_hbm, ex_hbm, out_hbm, mask_v, vpos_v, idx_v, rows_v,
                g0, g1, g2, g3, w0, w1, w2, w3):
    cid = lax.axis_index("c")
    sid = lax.axis_index("s")
    wid = sid * 2 + cid
    b = wid // 2
    h = wid % 2

    pltpu.sync_copy(mask_hbm.at[pl.ds(b * L, L)], mask_v)

    def _compact(i, cnt):
        mi = mask_v[pl.ds(i * LANES, LANES)]
        rank = cnt + plsc.cumsum(mi) - 1
        pos = i * LANES + lax.iota(jnp.int32, LANES)
        plsc.store_scatter(vpos_v, [rank], pos, mask=mi != 0)
        return cnt + jnp.sum(mi)

    nv = lax.fori_loop(0, L // LANES, _compact, jnp.int32(0))

    base_j = h * RPW
    base_g = b * L

    def _indices(k, _):
        jv = base_j + k * LANES + lax.iota(jnp.int32, LANES)
        src = plsc.load_gather(vpos_v, [jv % nv])
        idx_v[pl.ds(k * LANES, LANES)] = src + base_g
        return 0

    out0 = base_g + base_j
    gsems = (g0, g1, g2)
    wsems = (w0, w1, w2)

    def _gather(c):
        return pltpu.async_copy(
            ex_hbm.at[idx_v.at[pl.ds(c * CHUNK, CHUNK)]], rows_v.at[c % NBUF],
            gsems[c % NBUF])

    def _write(c):
        return pltpu.async_copy(
            rows_v.at[c % NBUF], out_hbm.at[pl.ds(out0 + c * CHUNK, CHUNK)],
            wsems[c % NBUF])

    lax.fori_loop(0, NBUF * CHUNK // LANES, _indices, 0)
    gd = {c: _gather(c) for c in range(NBUF)}
    lax.fori_loop(NBUF * CHUNK // LANES, RPW // LANES, _indices, 0)

    wd = {}
    for c in range(NCH):
        gd[c].wait()
        wd[c] = _write(c)
        if c + NBUF < NCH:
            wd[c].wait()
            gd[c + NBUF] = _gather(c + NBUF)
    for c in range(NCH - NBUF, NCH):
        wd[c].wait()


_CTX_TB = 2048


def _ctx_body(ctx_ref, out_ref):
    row = ctx_ref[pl.ds(pl.program_id(0), 1), :]
    out_ref[...] = jnp.broadcast_to(row, out_ref.shape)


_ctx_broadcast = pl.pallas_call(
    _ctx_body,
    grid=(B, L // _CTX_TB),
    in_specs=[pl.BlockSpec((B, D), lambda b, t: (0, 0))],
    out_specs=pl.BlockSpec((_CTX_TB, D), lambda b, t: (b * (L // _CTX_TB) + t, 0)),
    out_shape=jax.ShapeDtypeStruct((B * L, D), jnp.float32),
)


def kernel(context_feature, example_feature, mask):
    mask_i = mask.astype(jnp.int32).reshape(B * L)
    ex_flat = example_feature.reshape(B * L, D)
    flat_ex = _sc_flatten(mask_i, ex_flat)
    flat_ctx = _ctx_broadcast(context_feature)
    return flat_ctx, flat_ex


# speedup vs baseline: 1.0054x; 1.0054x over previous
import functools

import jax
import jax.numpy as jnp
from jax import lax
from jax.experimental import pallas as pl
from jax.experimental.pallas import tpu as pltpu
from jax.experimental.pallas import tpu_sc as plsc

B, L, D = 16, 2048, 256
LANES = 16
NW = 32
RPW = (B * L) // NW
CHUNK = 64
NCH = RPW // CHUNK
NBUF = 4

_SC_MESH = plsc.VectorSubcoreMesh(core_axis_name="c", subcore_axis_name="s")


@functools.partial(
    pl.kernel,
    out_type=jax.ShapeDtypeStruct((B * L, D), jnp.float32),
    mesh=_SC_MESH,
    compiler_params=pltpu.CompilerParams(needs_layout_passes=False),
    scratch_types=[
        pltpu.VMEM((L,), jnp.int32),
        pltpu.VMEM((L,), jnp.int32),
        pltpu.VMEM((RPW,), jnp.int32),
        pltpu.VMEM((NBUF, CHUNK, D), jnp.float32),
        pltpu.SemaphoreType.DMA,
        pltpu.SemaphoreType.DMA,
        pltpu.SemaphoreType.DMA,
        pltpu.SemaphoreType.DMA,
        pltpu.SemaphoreType.DMA,
        pltpu.SemaphoreType.DMA,
        pltpu.SemaphoreType.DMA,
        pltpu.SemaphoreType.DMA,
    ],
)
def _sc_flatten(mask_hbm, ex_hbm, out_hbm, mask_v, vpos_v, idx_v, rows_v,
                g0, g1, g2, g3, w0, w1, w2, w3):
    cid = lax.axis_index("c")
    sid = lax.axis_index("s")
    wid = sid * 2 + cid
    b = wid // 2
    h = wid % 2

    pltpu.sync_copy(mask_hbm.at[pl.ds(b * L, L)], mask_v)

    def _compact(i, cnt):
        mi = mask_v[pl.ds(i * LANES, LANES)]
        rank = cnt + plsc.cumsum(mi) - 1
        pos = i * LANES + lax.iota(jnp.int32, LANES)
        plsc.store_scatter(vpos_v, [rank], pos, mask=mi != 0)
        return cnt + jnp.sum(mi)

    nv = lax.fori_loop(0, L // LANES, _compact, jnp.int32(0))

    base_j = h * RPW
    base_g = b * L

    def _indices(k, _):
        jv = base_j + k * LANES + lax.iota(jnp.int32, LANES)
        src = plsc.load_gather(vpos_v, [jv % nv])
        idx_v[pl.ds(k * LANES, LANES)] = src + base_g
        return 0

    out0 = base_g + base_j
    gsems = (g0, g1, g2, g3)
    wsems = (w0, w1, w2, w3)

    def _gather(c):
        return pltpu.async_copy(
            ex_hbm.at[idx_v.at[pl.ds(c * CHUNK, CHUNK)]], rows_v.at[c % NBUF],
            gsems[c % NBUF])

    def _write(c):
        return pltpu.async_copy(
            rows_v.at[c % NBUF], out_hbm.at[pl.ds(out0 + c * CHUNK, CHUNK)],
            wsems[c % NBUF])

    lax.fori_loop(0, NBUF * CHUNK // LANES, _indices, 0)
    gd = {c: _gather(c) for c in range(NBUF)}
    lax.fori_loop(NBUF * CHUNK // LANES, RPW // LANES, _indices, 0)

    wd = {}
    for c in range(NCH):
        gd[c].wait()
        wd[c] = _write(c)
        if c + NBUF < NCH:
            wd[c].wait()
            gd[c + NBUF] = _gather(c + NBUF)
    for c in range(NCH - NBUF, NCH):
        wd[c].wait()


_CTX_TB = 2048


def _ctx_body(ctx_ref, out_ref):
    row = ctx_ref[pl.ds(pl.program_id(0), 1), :]
    out_ref[...] = jnp.broadcast_to(row, out_ref.shape)


_ctx_broadcast = pl.pallas_call(
    _ctx_body,
    grid=(B, L // _CTX_TB),
    in_specs=[pl.BlockSpec((B, D), lambda b, t: (0, 0))],
    out_specs=pl.BlockSpec((_CTX_TB, D), lambda b, t: (b * (L // _CTX_TB) + t, 0)),
    out_shape=jax.ShapeDtypeStruct((B * L, D), jnp.float32),
)


def kernel(context_feature, example_feature, mask):
    mask_i = mask.astype(jnp.int32).reshape(B * L)
    ex_flat = example_feature.reshape(B * L, D)
    flat_ex = _sc_flatten(mask_i, ex_flat)
    flat_ctx = _ctx_broadcast(context_feature)
    return flat_ctx, flat_ex
